# baseline (speedup 1.0000x reference)
import functools

import jax
import jax.numpy as jnp
from jax import lax
from jax.experimental import pallas as pl
from jax.experimental.pallas import tpu as pltpu
from jax.experimental.pallas import tpu_sc as plsc

VOCAB = 1000000
D = 64
B = 16384
C = 5

NW = 32
NWIN = VOCAB // 128
TAILV = NWIN * 128
TAIL = VOCAB - TAILV

NC_PAIRS = B * C // NW
NT_PAIRS = B // NW
NBUF = 4

CHUNK = 64
NCHUNK = B // NW // CHUNK


def _extract_phase(tab_hbm, tail_v, ids_buf, perm, nids, winid, winstart,
                   winb, stage0, stage1, sidx0, sidx1, out_hbm, wsems, ssems,
                   lane):
    nv16 = nids // 16

    def p1_body(jv, cursor):
        j0 = jv * 16
        w16 = ids_buf[pl.ds(j0 + 16, 16)] >> 7
        prev = ids_buf[pl.ds(j0 + 15, 16)] >> 7
        newwin = w16 != prev
        plsc.store_compressed(winid.at[pl.ds(cursor, 16)], w16, mask=newwin)
        plsc.store_compressed(winstart.at[pl.ds(cursor, 16)], j0 + lane,
                              mask=newwin)
        n = plsc.all_reduce_population_count(newwin)
        return cursor + n[0]

    nwin = lax.fori_loop(0, nv16, p1_body, jnp.int32(0))
    plsc.store_compressed(winstart.at[pl.ds(nwin, 16)],
                          jnp.full((16,), nids, jnp.int32), mask=lane == 0)

    def issue(i, b):
        @pl.when(i < nwin)
        def _():
            w = jnp.minimum(winid[pl.ds(i, 16)][0], NWIN - 1)
            off = pl.multiple_of(w * 128, 128)
            pltpu.async_copy(tab_hbm.at[:, :, pl.ds(off, 128)],
                             winb.at[b], wsems[b])

    def wait_win(i, b):
        @pl.when(i < nwin)
        def _():
            pltpu.make_async_copy(tab_hbm.at[:, :, pl.ds(0, 128)],
                                  winb.at[b], wsems[b]).wait()

    for b in range(NBUF):
        issue(jnp.int32(b), b)

    qidx = [((q * 16 + lane) >> 3, (q * 16 + lane) & 7) for q in range(4)]

    def extract_pairs(wb, j_lo, j_hi, st):
        def one(j, s):
            nst, fp, fc0, fc1, sv0, sv1 = s
            v = ids_buf[pl.ds(j + 16, 16)][0]
            slot = perm[pl.ds(j, 16)][0]
            lvec = jnp.full((16,), v & 127, jnp.int32)
            tail = v >= TAILV
            vrow = jnp.maximum(v - TAILV, 0)
            cols = []
            for q in range(4):
                main = plsc.load_gather(
                    winb.at[wb], [qidx[q][0], qidx[q][1], lvec])
                alt = tail_v[vrow, pl.ds(q * 16, 16)]
                cols.append(jnp.where(tail, alt, main))

            onehot = lane == nst
            sv0 = jnp.where(jnp.logical_and(fp == 0, onehot), slot, sv0)
            sv1 = jnp.where(jnp.logical_and(fp == 1, onehot), slot, sv1)

            @pl.when(fp == 0)
            def _():
                for q in range(4):
                    stage0[nst, pl.ds(q * 16, 16)] = cols[q]

            @pl.when(fp == 1)
            def _():
                for q in range(4):
                    stage1[nst, pl.ds(q * 16, 16)] = cols[q]

            full = nst == 15

            @pl.when(jnp.logical_and(full, fp == 0))
            def _():
                @pl.when(fc0 > 0)
                def _():
                    pltpu.make_async_copy(stage0, out_hbm.at[sidx0.at[0]],
                                          ssems[0]).wait()
                sidx0[0, pl.ds(0, 16)] = sv0
                pltpu.async_copy(stage0, out_hbm.at[sidx0.at[0]], ssems[0])

            @pl.when(jnp.logical_and(full, fp == 1))
            def _():
                @pl.when(fc1 > 0)
                def _():
                    pltpu.make_async_copy(stage1, out_hbm.at[sidx1.at[0]],
                                          ssems[1]).wait()
                sidx1[0, pl.ds(0, 16)] = sv1
                pltpu.async_copy(stage1, out_hbm.at[sidx1.at[0]], ssems[1])

            nst2 = jnp.where(full, 0, nst + 1)
            fp2 = jnp.where(full, 1 - fp, fp)
            fc0_2 = jnp.where(jnp.logical_and(full, fp == 0), fc0 + 1, fc0)
            fc1_2 = jnp.where(jnp.logical_and(full, fp == 1), fc1 + 1, fc1)
            return (nst2, fp2, fc0_2, fc1_2, sv0, sv1)

        return lax.fori_loop(j_lo, j_hi, one, st)

    def ring_body(g, st):
        for b in range(NBUF):
            i = g * NBUF + b
            wait_win(i, b)
            ws = winstart[pl.ds(i, 16)]
            st = lax.cond(
                i < nwin,
                lambda s, ws=ws, bb=b: extract_pairs(bb, ws[0], ws[1], s),
                lambda s: s, st)
            issue(i + NBUF, b)
        return st

    zero = jnp.int32(0)
    zvec = jnp.zeros((16,), jnp.int32)
    ngroups = (nwin + NBUF - 1) // NBUF
    st = lax.fori_loop(0, ngroups, ring_body,
                       (zero, zero, zero, zero, zvec, zvec))
    _, _, fc0, fc1, _, _ = st

    @pl.when(fc0 > 0)
    def _():
        pltpu.make_async_copy(stage0, out_hbm.at[sidx0.at[0]], ssems[0]).wait()

    @pl.when(fc1 > 0)
    def _():
        pltpu.make_async_copy(stage1, out_hbm.at[sidx1.at[0]], ssems[1]).wait()


def _make_k1():
    mesh = plsc.VectorSubcoreMesh(core_axis_name="c", subcore_axis_name="s")

    @functools.partial(
        pl.kernel,
        mesh=mesh,
        out_type=(jax.ShapeDtypeStruct((B, 128), jnp.float32),
                  jax.ShapeDtypeStruct((B * C, 128), jnp.float32)),
        scratch_types=[
            pltpu.VMEM((NT_PAIRS + 32,), jnp.int32),
            pltpu.VMEM((NT_PAIRS + 16,), jnp.int32),
            pltpu.VMEM((NC_PAIRS + 32,), jnp.int32),
            pltpu.VMEM((NC_PAIRS + 16,), jnp.int32),
            pltpu.VMEM((NC_PAIRS // 128, 128), jnp.int32),
            pltpu.VMEM((NC_PAIRS + 16,), jnp.int32),
            pltpu.VMEM((NC_PAIRS + 32,), jnp.int32),
            pltpu.VMEM((NBUF, 8, 8, 128), jnp.float32),
            pltpu.VMEM((TAIL, D), jnp.float32),
            pltpu.VMEM((TAIL, D), jnp.float32),
            pltpu.VMEM((16, 128), jnp.float32),
            pltpu.VMEM((16, 128), jnp.float32),
            pltpu.VMEM((1, 16), jnp.int32),
            pltpu.VMEM((1, 16), jnp.int32),
            pltpu.SemaphoreType.DMA,
            pltpu.SemaphoreType.DMA,
            pltpu.SemaphoreType.DMA,
            pltpu.SemaphoreType.DMA,
            pltpu.SemaphoreType.DMA,
            pltpu.SemaphoreType.DMA,
        ],
        compiler_params=pltpu.CompilerParams(needs_layout_passes=False),
    )
    def k1(tid_hbm, tperm_hbm, cid_hbm, cperm_hbm, ttabT, ctabT,
           ttail_hbm, ctail_hbm, tout_hbm, cout_hbm,
           tids, tpm, cids, cpm, idstg, winid, winstart, winb, ttail, ctail,
           stage0, stage1, sidx0, sidx1,
           w0, w1, w2, w3, ss0, ss1):
        wid = lax.axis_index("s") * 2 + lax.axis_index("c")
        lane = lax.iota(jnp.int32, 16)
        wsems = (w0, w1, w2, w3)
        ssems = (ss0, ss1)

        def spread(hbm3d, dst1d, off, nrows):
            pltpu.sync_copy(hbm3d.at[wid], idstg.at[pl.ds(0, nrows)])
            for r in range(nrows):
                for q in range(8):
                    dst1d[pl.ds(off + r * 128 + q * 16, 16)] = (
                        idstg[r, pl.ds(q * 16, 16)])

        spread(tid_hbm, tids, 16, NT_PAIRS // 128)
        spread(tperm_hbm, tpm, 0, NT_PAIRS // 128)
        spread(cid_hbm, cids, 16, NC_PAIRS // 128)
        spread(cperm_hbm, cpm, 0, NC_PAIRS // 128)
        pltpu.sync_copy(ttail_hbm, ttail)
        pltpu.sync_copy(ctail_hbm, ctail)
        sent = jnp.full((16,), -1, jnp.int32)
        pos15 = jnp.full((16,), 15, jnp.int32)
        plsc.store_scatter(tids, [pos15], sent, mask=lane == 0)
        plsc.store_scatter(cids, [pos15], sent, mask=lane == 0)

        _extract_phase(ttabT, ttail, tids, tpm, NT_PAIRS, winid, winstart,
                       winb, stage0, stage1, sidx0, sidx1, tout_hbm,
                       wsems, ssems, lane)
        _extract_phase(ctabT, ctail, cids, cpm, NC_PAIRS, winid, winstart,
                       winb, stage0, stage1, sidx0, sidx1, cout_hbm,
                       wsems, ssems, lane)

    return k1


def _make_k2():
    mesh = plsc.VectorSubcoreMesh(core_axis_name="c", subcore_axis_name="s")

    @functools.partial(
        pl.kernel,
        mesh=mesh,
        out_type=jax.ShapeDtypeStruct((B * C,), jnp.float32),
        scratch_types=[
            pltpu.VMEM((CHUNK, 128), jnp.float32),
            pltpu.VMEM((CHUNK, 128), jnp.float32),
            pltpu.VMEM((CHUNK * C, 128), jnp.float32),
            pltpu.VMEM((CHUNK * C, 128), jnp.float32),
            pltpu.VMEM((CHUNK * C,), jnp.float32),
            pltpu.SemaphoreType.DMA,
            pltpu.SemaphoreType.DMA,
            pltpu.SemaphoreType.DMA,
            pltpu.SemaphoreType.DMA,
        ],
        compiler_params=pltpu.CompilerParams(needs_layout_passes=False),
    )
    def k2(trows_hbm, crows_hbm, out_hbm, rt0, rt1, rc0, rc1, ov,
           st0, st1, sc0, sc1):
        wid = lax.axis_index("s") * 2 + lax.axis_index("c")
        rts, rcs = (rt0, rt1), (rc0, rc1)
        sts, scs = (st0, st1), (sc0, sc1)
        base_b = wid * (B // NW)

        def issue(k):
            buf = k % 2
            cps = [pltpu.async_copy(
                trows_hbm.at[pl.ds(base_b + k * CHUNK, CHUNK)],
                rts[buf], sts[buf])]
            cps.append(pltpu.async_copy(
                crows_hbm.at[pl.ds((base_b + k * CHUNK) * C, CHUNK * C)],
                rcs[buf], scs[buf]))
            return cps

        lane = lax.iota(jnp.int32, 16)

        def compute(rt, rc, k):
            for g in range(CHUNK // 16):
                rowb = g * 16 + lane
                crows = [g * 16 * C + lane * C + c for c in range(C)]

                def body(d, accs):
                    dcol = jnp.full((16,), d, jnp.int32)
                    t = plsc.load_gather(rt, [rowb, dcol])
                    return tuple(
                        accs[c] + t * plsc.load_gather(rc, [crows[c], dcol])
                        for c in range(C))

                accs = lax.fori_loop(
                    0, D, body,
                    tuple(jnp.zeros((16,), jnp.float32) for _ in range(C)))
                for c in range(C):
                    plsc.store_scatter(ov, [crows[c]], accs[c])
            pltpu.sync_copy(ov, out_hbm.at[pl.ds((base_b + k * CHUNK) * C,
                                                 CHUNK * C)])

        inflight = issue(0)
        for k in range(NCHUNK):
            for cp in inflight:
                cp.wait()
            if k + 1 < NCHUNK:
                inflight = issue(k + 1)
            compute(rts[k % 2], rcs[k % 2], k)

    return k2


_K1 = _make_k1()
_K2 = _make_k2()


@jax.jit
def kernel(target, context, target_table, context_table):
    tgt = target.astype(jnp.int32)
    ctx = context.astype(jnp.int32).reshape(-1)

    tperm = jnp.argsort(tgt).astype(jnp.int32)
    tsort = jnp.take(tgt, tperm)
    cperm = jnp.argsort(ctx).astype(jnp.int32)
    csort = jnp.take(ctx, cperm)

    ttabT = target_table.T.reshape(8, 8, VOCAB)
    ctabT = context_table.T.reshape(8, 8, VOCAB)
    ttail = target_table[TAILV:, :]
    ctail = context_table[TAILV:, :]

    trows, crows = _K1(
        tsort.reshape(NW, NT_PAIRS // 128, 128),
        tperm.reshape(NW, NT_PAIRS // 128, 128),
        csort.reshape(NW, NC_PAIRS // 128, 128),
        cperm.reshape(NW, NC_PAIRS // 128, 128),
        ttabT, ctabT, ttail, ctail)
    flat = _K2(trows, crows)
    return flat.reshape(B, C)


# speedup vs baseline: 1.0306x; 1.0306x over previous
import functools

import jax
import jax.numpy as jnp
from jax import lax
from jax.experimental import pallas as pl
from jax.experimental.pallas import tpu as pltpu
from jax.experimental.pallas import tpu_sc as plsc

VOCAB = 1000000
D = 64
B = 16384
C = 5

NW = 32
WINW = 256
NWIN = VOCAB // WINW
TAILV = NWIN * WINW
TAIL = VOCAB - TAILV

NC_PAIRS = B * C // NW
NT_PAIRS = B // NW
NBUF = 4

CHUNK = 64
NCHUNK = B // NW // CHUNK


def _extract_phase(tab_hbm, tail_v, ids_buf, perm, nids, winid, winstart,
                   winb, stage0, stage1, sidx0, sidx1, out_hbm, wsems, ssems,
                   lane):
    nv16 = nids // 16

    def p1_body(jv, cursor):
        j0 = jv * 16
        w16 = ids_buf[pl.ds(j0 + 16, 16)] >> 8
        prev = ids_buf[pl.ds(j0 + 15, 16)] >> 8
        newwin = w16 != prev
        plsc.store_compressed(winid.at[pl.ds(cursor, 16)], w16, mask=newwin)
        plsc.store_compressed(winstart.at[pl.ds(cursor, 16)], j0 + lane,
                              mask=newwin)
        n = plsc.all_reduce_population_count(newwin)
        return cursor + n[0]

    nwin = lax.fori_loop(0, nv16, p1_body, jnp.int32(0))
    plsc.store_compressed(winstart.at[pl.ds(nwin, 16)],
                          jnp.full((16,), nids, jnp.int32), mask=lane == 0)

    def issue(i, b):
        @pl.when(i < nwin)
        def _():
            w = jnp.minimum(winid[pl.ds(i, 16)][0], NWIN - 1)
            off = pl.multiple_of(w * WINW, WINW)
            pltpu.async_copy(tab_hbm.at[:, :, pl.ds(off, WINW)],
                             winb.at[b], wsems[b])

    def wait_win(i, b):
        @pl.when(i < nwin)
        def _():
            pltpu.make_async_copy(tab_hbm.at[:, :, pl.ds(0, WINW)],
                                  winb.at[b], wsems[b]).wait()

    for b in range(NBUF):
        issue(jnp.int32(b), b)

    qidx = [((q * 16 + lane) >> 3, (q * 16 + lane) & 7) for q in range(4)]

    def extract_pairs(wb, j_lo, j_hi, st):
        def one(j, s):
            nst, fp, fc0, fc1, sv0, sv1 = s
            v = ids_buf[pl.ds(j + 16, 16)][0]
            slot = perm[pl.ds(j, 16)][0]
            lvec = jnp.full((16,), v & (WINW - 1), jnp.int32)
            tail = v >= TAILV
            vrow = jnp.maximum(v - TAILV, 0)
            cols = []
            for q in range(4):
                main = plsc.load_gather(
                    winb.at[wb], [qidx[q][0], qidx[q][1], lvec])
                alt = tail_v[vrow, pl.ds(q * 16, 16)]
                cols.append(jnp.where(tail, alt, main))

            onehot = lane == nst
            sv0 = jnp.where(jnp.logical_and(fp == 0, onehot), slot, sv0)
            sv1 = jnp.where(jnp.logical_and(fp == 1, onehot), slot, sv1)

            @pl.when(fp == 0)
            def _():
                for q in range(4):
                    stage0[nst, pl.ds(q * 16, 16)] = cols[q]

            @pl.when(fp == 1)
            def _():
                for q in range(4):
                    stage1[nst, pl.ds(q * 16, 16)] = cols[q]

            full = nst == 15

            @pl.when(jnp.logical_and(full, fp == 0))
            def _():
                @pl.when(fc0 > 0)
                def _():
                    pltpu.make_async_copy(stage0, out_hbm.at[sidx0.at[0]],
                                          ssems[0]).wait()
                sidx0[0, pl.ds(0, 16)] = sv0
                pltpu.async_copy(stage0, out_hbm.at[sidx0.at[0]], ssems[0])

            @pl.when(jnp.logical_and(full, fp == 1))
            def _():
                @pl.when(fc1 > 0)
                def _():
                    pltpu.make_async_copy(stage1, out_hbm.at[sidx1.at[0]],
                                          ssems[1]).wait()
                sidx1[0, pl.ds(0, 16)] = sv1
                pltpu.async_copy(stage1, out_hbm.at[sidx1.at[0]], ssems[1])

            nst2 = jnp.where(full, 0, nst + 1)
            fp2 = jnp.where(full, 1 - fp, fp)
            fc0_2 = jnp.where(jnp.logical_and(full, fp == 0), fc0 + 1, fc0)
            fc1_2 = jnp.where(jnp.logical_and(full, fp == 1), fc1 + 1, fc1)
            return (nst2, fp2, fc0_2, fc1_2, sv0, sv1)

        return lax.fori_loop(j_lo, j_hi, one, st)

    def ring_body(g, st):
        for b in range(NBUF):
            i = g * NBUF + b
            wait_win(i, b)
            ws = winstart[pl.ds(i, 16)]
            st = lax.cond(
                i < nwin,
                lambda s, ws=ws, bb=b: extract_pairs(bb, ws[0], ws[1], s),
                lambda s: s, st)
            issue(i + NBUF, b)
        return st

    zero = jnp.int32(0)
    zvec = jnp.zeros((16,), jnp.int32)
    ngroups = (nwin + NBUF - 1) // NBUF
    st = lax.fori_loop(0, ngroups, ring_body,
                       (zero, zero, zero, zero, zvec, zvec))
    _, _, fc0, fc1, _, _ = st

    @pl.when(fc0 > 0)
    def _():
        pltpu.make_async_copy(stage0, out_hbm.at[sidx0.at[0]], ssems[0]).wait()

    @pl.when(fc1 > 0)
    def _():
        pltpu.make_async_copy(stage1, out_hbm.at[sidx1.at[0]], ssems[1]).wait()


def _make_k1(npairs):
    mesh = plsc.VectorSubcoreMesh(core_axis_name="c", subcore_axis_name="s")

    @functools.partial(
        pl.kernel,
        mesh=mesh,
        out_type=jax.ShapeDtypeStruct((npairs * NW, 128), jnp.float32),
        scratch_types=[
            pltpu.VMEM((npairs + 32,), jnp.int32),
            pltpu.VMEM((npairs + 16,), jnp.int32),
            pltpu.VMEM((npairs // 128, 128), jnp.int32),
            pltpu.VMEM((npairs // 128, 128), jnp.int32),
            pltpu.VMEM((npairs + 16,), jnp.int32),
            pltpu.VMEM((npairs + 32,), jnp.int32),
            pltpu.VMEM((NBUF, 8, 8, WINW), jnp.float32),
            pltpu.VMEM((TAIL, D), jnp.float32),
            pltpu.VMEM((16, 128), jnp.float32),
            pltpu.VMEM((16, 128), jnp.float32),
            pltpu.VMEM((1, 16), jnp.int32),
            pltpu.VMEM((1, 16), jnp.int32),
            pltpu.SemaphoreType.DMA,
            pltpu.SemaphoreType.DMA,
            pltpu.SemaphoreType.DMA,
            pltpu.SemaphoreType.DMA,
            pltpu.SemaphoreType.DMA,
            pltpu.SemaphoreType.DMA,
        ],
        compiler_params=pltpu.CompilerParams(needs_layout_passes=False),
    )
    def k1(flat_hbm, perm_hbm, tabT, tail_hbm, out_hbm,
           ids, pm, idstg, sidstg, winid, winstart, winb, tail_v,
           stage0, stage1, sidx0, sidx1,
           w0, w1, w2, w3, ss0, ss1):
        wid = lax.axis_index("s") * 2 + lax.axis_index("c")
        lane = lax.iota(jnp.int32, 16)
        wsems = (w0, w1, w2, w3)
        ssems = (ss0, ss1)

        nrows = npairs // 128
        pltpu.sync_copy(perm_hbm.at[wid], idstg.at[pl.ds(0, nrows)])
        cps = [pltpu.async_copy(flat_hbm.at[idstg.at[r]], sidstg.at[r], w0)
               for r in range(nrows)]
        for cp in cps:
            cp.wait()
        for r in range(nrows):
            for q in range(8):
                pm[pl.ds(r * 128 + q * 16, 16)] = (
                    idstg[r, pl.ds(q * 16, 16)])
                ids[pl.ds(16 + r * 128 + q * 16, 16)] = (
                    sidstg[r, pl.ds(q * 16, 16)])
        pltpu.sync_copy(tail_hbm, tail_v)
        sent = jnp.full((16,), -1, jnp.int32)
        pos15 = jnp.full((16,), 15, jnp.int32)
        plsc.store_scatter(ids, [pos15], sent, mask=lane == 0)

        _extract_phase(tabT, tail_v, ids, pm, npairs, winid, winstart,
                       winb, stage0, stage1, sidx0, sidx1, out_hbm,
                       wsems, ssems, lane)

    return k1


def _make_k2():
    mesh = plsc.VectorSubcoreMesh(core_axis_name="c", subcore_axis_name="s")

    @functools.partial(
        pl.kernel,
        mesh=mesh,
        out_type=jax.ShapeDtypeStruct((B * C,), jnp.float32),
        scratch_types=[
            pltpu.VMEM((CHUNK, 128), jnp.float32),
            pltpu.VMEM((CHUNK, 128), jnp.float32),
            pltpu.VMEM((CHUNK * C, 128), jnp.float32),
            pltpu.VMEM((CHUNK * C, 128), jnp.float32),
            pltpu.VMEM((CHUNK * C,), jnp.float32),
            pltpu.SemaphoreType.DMA,
            pltpu.SemaphoreType.DMA,
            pltpu.SemaphoreType.DMA,
            pltpu.SemaphoreType.DMA,
        ],
        compiler_params=pltpu.CompilerParams(needs_layout_passes=False),
    )
    def k2(trows_hbm, crows_hbm, out_hbm, rt0, rt1, rc0, rc1, ov,
           st0, st1, sc0, sc1):
        wid = lax.axis_index("s") * 2 + lax.axis_index("c")
        rts, rcs = (rt0, rt1), (rc0, rc1)
        sts, scs = (st0, st1), (sc0, sc1)
        base_b = wid * (B // NW)

        def issue(k):
            buf = k % 2
            cps = [pltpu.async_copy(
                trows_hbm.at[pl.ds(base_b + k * CHUNK, CHUNK)],
                rts[buf], sts[buf])]
            cps.append(pltpu.async_copy(
                crows_hbm.at[pl.ds((base_b + k * CHUNK) * C, CHUNK * C)],
                rcs[buf], scs[buf]))
            return cps

        lane = lax.iota(jnp.int32, 16)

        def compute(rt, rc, k):
            for g in range(CHUNK // 16):
                rowb = g * 16 + lane
                crows = [g * 16 * C + lane * C + c for c in range(C)]

                def body(d, accs):
                    dcol = jnp.full((16,), d, jnp.int32)
                    t = plsc.load_gather(rt, [rowb, dcol])
                    return tuple(
                        accs[c] + t * plsc.load_gather(rc, [crows[c], dcol])
                        for c in range(C))

                accs = plsc.parallel_loop(
                    0, D, unroll=8,
                    carry=tuple(jnp.zeros((16,), jnp.float32)
                                for _ in range(C)))(body)
                for c in range(C):
                    plsc.store_scatter(ov, [crows[c]], accs[c])
            pltpu.sync_copy(ov, out_hbm.at[pl.ds((base_b + k * CHUNK) * C,
                                                 CHUNK * C)])

        inflight = issue(0)
        for k in range(NCHUNK):
            for cp in inflight:
                cp.wait()
            if k + 1 < NCHUNK:
                inflight = issue(k + 1)
            compute(rts[k % 2], rcs[k % 2], k)

    return k2


_K1T = _make_k1(NT_PAIRS)
_K1C = _make_k1(NC_PAIRS)
_K2 = _make_k2()


@jax.jit
def kernel(target, context, target_table, context_table):
    tgt = target.astype(jnp.int32)
    ctx = context.astype(jnp.int32).reshape(-1)

    tperm = jnp.argsort(tgt).astype(jnp.int32)
    dep = (tperm[0] >> 31) & 1
    cperm = jnp.argsort(ctx + dep).astype(jnp.int32)

    ttabT = target_table.T.reshape(8, 8, VOCAB)
    ctabT = context_table.T.reshape(8, 8, VOCAB)
    ttail = target_table[TAILV:, :]
    ctail = context_table[TAILV:, :]

    trows = _K1T(tgt, tperm.reshape(NW, NT_PAIRS // 128, 128),
                 ttabT, ttail)
    crows = _K1C(ctx, cperm.reshape(NW, NC_PAIRS // 128, 128),
                 ctabT, ctail)
    flat = _K2(trows, crows)
    return flat.reshape(B, C)
